# baseline (speedup 1.0000x reference)
import functools

import jax
import jax.numpy as jnp
from jax import lax
from jax.experimental import pallas as pl
from jax.experimental.pallas import tpu as pltpu
from jax.experimental.pallas import tpu_sc as plsc

N_FEATURES = 128
WINDOW = 10
T = 4096
LEAD = 16
HP_ROWS = LEAD + T + 16
NW = 32
WIN_PER_W = T // NW
STAGE_ROWS = WIN_PER_W + 2 * LEAD


def _dense_body(ori_ref, w_ref, al_ref, ar_ref, bias_ref, hb_ref, el_ref, er_ref):
    h = jnp.dot(ori_ref[...], w_ref[...], preferred_element_type=jnp.float32)
    hb = h + bias_ref[...]
    hb_ref[pl.ds(LEAD, T)] = hb
    hb_ref[pl.ds(0, LEAD)] = jnp.broadcast_to(hb[0:1], (LEAD, N_FEATURES))
    hb_ref[pl.ds(LEAD + T, HP_ROWS - LEAD - T)] = jnp.zeros(
        (HP_ROWS - LEAD - T, N_FEATURES), jnp.float32)

    dn = (((1,), (1,)), ((), ()))
    el = lax.dot_general(al_ref[...], h, dn, precision=lax.Precision.HIGHEST,
                         preferred_element_type=jnp.float32)
    er = lax.dot_general(ar_ref[...], h, dn, precision=lax.Precision.HIGHEST,
                         preferred_element_type=jnp.float32)
    el_ref[:, pl.ds(LEAD, T)] = el
    el_ref[:, pl.ds(0, LEAD)] = jnp.broadcast_to(el[:, 0:1], (1, LEAD))
    el_ref[:, pl.ds(LEAD + T, HP_ROWS - LEAD - T)] = jnp.zeros(
        (1, HP_ROWS - LEAD - T), jnp.float32)
    er_ref[:, pl.ds(LEAD, T)] = er
    er_ref[:, pl.ds(0, LEAD)] = jnp.broadcast_to(er[:, 0:1], (1, LEAD))
    er_ref[:, pl.ds(LEAD + T, HP_ROWS - LEAD - T)] = jnp.zeros(
        (1, HP_ROWS - LEAD - T), jnp.float32)


def _sc_agg_body(hb_hbm, el_hbm, er_hbm, out_hbm, h_v, el_v, er_v, alpha_v, out_v, sem):
    wid = lax.axis_index("s") * 2 + lax.axis_index("c")
    base = wid * WIN_PER_W

    pltpu.sync_copy(hb_hbm.at[pl.ds(base, STAGE_ROWS)], h_v)
    pltpu.sync_copy(el_hbm.at[pl.ds(base, STAGE_ROWS)], el_v)
    pltpu.sync_copy(er_hbm.at[pl.ds(base + LEAD, WIN_PER_W)], er_v)

    for g in range(WIN_PER_W // 16):
        t0 = g * 16
        er9 = er_v[pl.ds(t0, 16)]
        scores = []
        for i in range(WINDOW):
            s = el_v[pl.ds(t0 + i + 7, 16)] + er9
            scores.append(jnp.where(s > 0, s, 0.2 * s))
        m = scores[0]
        for i in range(1, WINDOW):
            m = jnp.maximum(m, scores[i])
        ees = [jnp.exp(s - m) for s in scores]
        denom = ees[0]
        for i in range(1, WINDOW):
            denom = denom + ees[i]
        inv = 1.0 / denom
        for i in range(WINDOW):
            alpha_v[i, pl.ds(t0, 16)] = ees[i] * inv

    @plsc.parallel_loop(0, WIN_PER_W // 16, 1)
    def body(g):
        t0 = g * 16
        av = [alpha_v[i, pl.ds(t0, 16)] for i in range(WINDOW)]
        ab = [[av[i].at[jnp.full((16,), tt, jnp.int32)].get(
                  mode="promise_in_bounds") for i in range(WINDOW)]
              for tt in range(16)]
        for c in range(N_FEATURES // 16):
            rows = [h_v[t0 + r + 7, pl.ds(c * 16, 16)]
                    for r in range(16 + WINDOW - 1)]
            for tt in range(16):
                acc = ab[tt][0] * rows[tt]
                for i in range(1, WINDOW):
                    acc = acc + ab[tt][i] * rows[tt + i]
                out_v[t0 + tt, pl.ds(c * 16, 16)] = acc

    pltpu.sync_copy(out_v, out_hbm.at[pl.ds(base, WIN_PER_W)])


_sc_agg = functools.partial(
    pl.kernel,
    out_type=jax.ShapeDtypeStruct((T, N_FEATURES), jnp.float32),
    mesh=plsc.VectorSubcoreMesh(core_axis_name="c", subcore_axis_name="s"),
    scratch_types=[
        pltpu.VMEM((STAGE_ROWS, N_FEATURES), jnp.float32),
        pltpu.VMEM((STAGE_ROWS,), jnp.float32),
        pltpu.VMEM((WIN_PER_W,), jnp.float32),
        pltpu.VMEM((WINDOW, WIN_PER_W), jnp.float32),
        pltpu.VMEM((WIN_PER_W, N_FEATURES), jnp.float32),
        pltpu.SemaphoreType.DMA,
    ],
)(_sc_agg_body)


def kernel(ori_feats, W, attn_l, attn_r, bias):
    hb, el, er = pl.pallas_call(
        _dense_body,
        out_shape=[
            jax.ShapeDtypeStruct((HP_ROWS, N_FEATURES), jnp.float32),
            jax.ShapeDtypeStruct((1, HP_ROWS), jnp.float32),
            jax.ShapeDtypeStruct((1, HP_ROWS), jnp.float32),
        ],
        in_specs=[pl.BlockSpec(memory_space=pltpu.VMEM)] * 5,
        out_specs=[pl.BlockSpec(memory_space=pltpu.VMEM)] * 3,
    )(ori_feats, W, attn_l.reshape(1, N_FEATURES), attn_r.reshape(1, N_FEATURES),
      bias.reshape(1, N_FEATURES))

    out = _sc_agg(hb, el.reshape(HP_ROWS), er.reshape(HP_ROWS))
    return out[:, None, :]


# speedup vs baseline: 1.3023x; 1.3023x over previous
import functools

import jax
import jax.numpy as jnp
from jax import lax
from jax.experimental import pallas as pl
from jax.experimental.pallas import tpu as pltpu
from jax.experimental.pallas import tpu_sc as plsc

N_FEATURES = 128
WINDOW = 10
T = 4096
LEAD = 16
HP_ROWS = LEAD + T + 16
NW = 32
WIN_PER_W = T // NW
STAGE_ROWS = WIN_PER_W + 2 * LEAD


def _dense_body(ori_ref, w_ref, alr_ref, bias_ref, hb_ref, el_ref, er_ref):
    h = jnp.dot(ori_ref[...], w_ref[...], preferred_element_type=jnp.float32)
    hb = h + bias_ref[...]
    hb_ref[pl.ds(LEAD, T)] = hb
    hb_ref[pl.ds(0, LEAD)] = jnp.broadcast_to(hb[0:1], (LEAD, N_FEATURES))
    hb_ref[pl.ds(LEAD + T, HP_ROWS - LEAD - T)] = jnp.zeros(
        (HP_ROWS - LEAD - T, N_FEATURES), jnp.float32)

    dn = (((1,), (1,)), ((), ()))
    elr = lax.dot_general(alr_ref[...], h, dn, precision=lax.Precision.HIGHEST,
                          preferred_element_type=jnp.float32)
    zeros_tail = jnp.zeros((1, HP_ROWS - LEAD - T), jnp.float32)
    el = elr[0:1]
    er = elr[1:2]
    el_ref[pl.ds(LEAD, T)] = el.reshape(T)
    el_ref[pl.ds(0, LEAD)] = jnp.broadcast_to(el[:, 0:1], (1, LEAD)).reshape(LEAD)
    el_ref[pl.ds(LEAD + T, HP_ROWS - LEAD - T)] = zeros_tail.reshape(-1)
    er_ref[pl.ds(LEAD, T)] = er.reshape(T)
    er_ref[pl.ds(0, LEAD)] = jnp.broadcast_to(er[:, 0:1], (1, LEAD)).reshape(LEAD)
    er_ref[pl.ds(LEAD + T, HP_ROWS - LEAD - T)] = zeros_tail.reshape(-1)


def _sc_agg_body(hb_hbm, el_hbm, er_hbm, out_hbm, h_v, el_v, er_v, alpha_v, out_v, sem):
    wid = lax.axis_index("s") * 2 + lax.axis_index("c")
    base = wid * WIN_PER_W

    pltpu.sync_copy(hb_hbm.at[pl.ds(base, STAGE_ROWS)], h_v)
    pltpu.sync_copy(el_hbm.at[pl.ds(base, STAGE_ROWS)], el_v)
    pltpu.sync_copy(er_hbm.at[pl.ds(base + LEAD, WIN_PER_W)], er_v)

    for g in range(WIN_PER_W // 16):
        t0 = g * 16
        er9 = er_v[pl.ds(t0, 16)]
        scores = []
        for i in range(WINDOW):
            s = el_v[pl.ds(t0 + i + 7, 16)] + er9
            scores.append(jnp.where(s > 0, s, 0.2 * s))
        m = scores[0]
        for i in range(1, WINDOW):
            m = jnp.maximum(m, scores[i])
        ees = [jnp.exp(s - m) for s in scores]
        denom = ees[0]
        for i in range(1, WINDOW):
            denom = denom + ees[i]
        inv = 1.0 / denom
        for i in range(WINDOW):
            alpha_v[i, pl.ds(t0, 16)] = ees[i] * inv

    @plsc.parallel_loop(0, WIN_PER_W // 16, 1)
    def body(g):
        t0 = g * 16
        av = [alpha_v[i, pl.ds(t0, 16)] for i in range(WINDOW)]
        for tb in range(4):
            ab = [[av[i].at[jnp.full((16,), tb * 4 + u, jnp.int32)].get(
                      mode="promise_in_bounds") for i in range(WINDOW)]
                  for u in range(4)]
            for c in range(N_FEATURES // 16):
                rows = [h_v[t0 + tb * 4 + r + 7, pl.ds(c * 16, 16)]
                        for r in range(4 + WINDOW - 1)]
                for u in range(4):
                    acc = ab[u][0] * rows[u]
                    for i in range(1, WINDOW):
                        acc = acc + ab[u][i] * rows[u + i]
                    out_v[t0 + tb * 4 + u, pl.ds(c * 16, 16)] = acc

    pltpu.sync_copy(out_v, out_hbm.at[pl.ds(base, WIN_PER_W)])


_sc_agg = functools.partial(
    pl.kernel,
    out_type=jax.ShapeDtypeStruct((T, N_FEATURES), jnp.float32),
    mesh=plsc.VectorSubcoreMesh(core_axis_name="c", subcore_axis_name="s"),
    scratch_types=[
        pltpu.VMEM((STAGE_ROWS, N_FEATURES), jnp.float32),
        pltpu.VMEM((STAGE_ROWS,), jnp.float32),
        pltpu.VMEM((WIN_PER_W,), jnp.float32),
        pltpu.VMEM((WINDOW, WIN_PER_W), jnp.float32),
        pltpu.VMEM((WIN_PER_W, N_FEATURES), jnp.float32),
        pltpu.SemaphoreType.DMA,
    ],
)(_sc_agg_body)


def kernel(ori_feats, W, attn_l, attn_r, bias):
    alr = jnp.stack([attn_l, attn_r], axis=0)
    hb, el, er = pl.pallas_call(
        _dense_body,
        out_shape=[
            jax.ShapeDtypeStruct((HP_ROWS, N_FEATURES), jnp.float32),
            jax.ShapeDtypeStruct((HP_ROWS,), jnp.float32),
            jax.ShapeDtypeStruct((HP_ROWS,), jnp.float32),
        ],
        in_specs=[pl.BlockSpec(memory_space=pltpu.VMEM)] * 4,
        out_specs=[pl.BlockSpec(memory_space=pltpu.VMEM)] * 3,
    )(ori_feats, W, alr, bias.reshape(1, N_FEATURES))

    out = _sc_agg(hb, el, er)
    return out[:, None, :]


# speedup vs baseline: 1.3990x; 1.0743x over previous
import functools

import jax
import jax.numpy as jnp
from jax import lax
from jax.experimental import pallas as pl
from jax.experimental.pallas import tpu as pltpu
from jax.experimental.pallas import tpu_sc as plsc

N_FEATURES = 128
WINDOW = 10
T = 4096
LEAD = 16
HP_ROWS = LEAD + T + 16
NW = 32
WIN_PER_W = T // NW
STAGE_ROWS = WIN_PER_W + 2 * LEAD


def _dense_body(ori_ref, w_ref, al_ref, ar_ref, bias_ref, hb_ref, el_ref, er_ref):
    h = jnp.dot(ori_ref[...], w_ref[...], preferred_element_type=jnp.float32)
    alr = jnp.concatenate([al_ref[...], ar_ref[...]], axis=0)
    hb = h + bias_ref[...]
    hb_ref[pl.ds(LEAD, T)] = hb
    hb_ref[pl.ds(0, LEAD)] = jnp.broadcast_to(hb[0:1], (LEAD, N_FEATURES))
    hb_ref[pl.ds(LEAD + T, HP_ROWS - LEAD - T)] = jnp.zeros(
        (HP_ROWS - LEAD - T, N_FEATURES), jnp.float32)

    dn = (((1,), (1,)), ((), ()))
    elr = lax.dot_general(alr, h, dn, precision=lax.Precision.HIGHEST,
                          preferred_element_type=jnp.float32)
    zeros_tail = jnp.zeros((1, HP_ROWS - LEAD - T), jnp.float32)
    el = elr[0:1]
    er = elr[1:2]
    el_ref[pl.ds(LEAD, T)] = el.reshape(T)
    el_ref[pl.ds(0, LEAD)] = jnp.broadcast_to(el[:, 0:1], (1, LEAD)).reshape(LEAD)
    el_ref[pl.ds(LEAD + T, HP_ROWS - LEAD - T)] = zeros_tail.reshape(-1)
    er_ref[pl.ds(LEAD, T)] = er.reshape(T)
    er_ref[pl.ds(0, LEAD)] = jnp.broadcast_to(er[:, 0:1], (1, LEAD)).reshape(LEAD)
    er_ref[pl.ds(LEAD + T, HP_ROWS - LEAD - T)] = zeros_tail.reshape(-1)


def _sc_agg_body(hb_hbm, el_hbm, er_hbm, out_hbm, h_v, el_v, er_v, alpha_v, out_v,
                 sem_h, sem_s):
    wid = lax.axis_index("s") * 2 + lax.axis_index("c")
    base = wid * WIN_PER_W

    h_copy = pltpu.async_copy(hb_hbm.at[pl.ds(base, STAGE_ROWS)], h_v, sem_h)
    pltpu.async_copy(el_hbm.at[pl.ds(base, STAGE_ROWS)], el_v, sem_s).wait()
    pltpu.async_copy(er_hbm.at[pl.ds(base + LEAD, WIN_PER_W)], er_v, sem_s).wait()

    for g in range(WIN_PER_W // 16):
        t0 = g * 16
        er9 = er_v[pl.ds(t0, 16)]
        scores = []
        for i in range(WINDOW):
            s = el_v[pl.ds(t0 + i + 7, 16)] + er9
            scores.append(jnp.where(s > 0, s, 0.2 * s))
        m = scores[0]
        for i in range(1, WINDOW):
            m = jnp.maximum(m, scores[i])
        ees = [jnp.exp(s - m) for s in scores]
        denom = ees[0]
        for i in range(1, WINDOW):
            denom = denom + ees[i]
        inv = 1.0 / denom
        for i in range(WINDOW):
            alpha_v[i, pl.ds(t0, 16)] = ees[i] * inv

    h_copy.wait()

    @plsc.parallel_loop(0, WIN_PER_W // 16, 1)
    def body(g):
        t0 = g * 16
        av = [alpha_v[i, pl.ds(t0, 16)] for i in range(WINDOW)]
        for tb in range(4):
            ab = [[av[i].at[jnp.full((16,), tb * 4 + u, jnp.int32)].get(
                      mode="promise_in_bounds") for i in range(WINDOW)]
                  for u in range(4)]
            for c in range(N_FEATURES // 16):
                rows = [h_v[t0 + tb * 4 + r + 7, pl.ds(c * 16, 16)]
                        for r in range(4 + WINDOW - 1)]
                for u in range(4):
                    acc = ab[u][0] * rows[u]
                    for i in range(1, WINDOW):
                        acc = acc + ab[u][i] * rows[u + i]
                    out_v[t0 + tb * 4 + u, pl.ds(c * 16, 16)] = acc

    pltpu.sync_copy(out_v, out_hbm.at[pl.ds(base, WIN_PER_W)])


_sc_agg = functools.partial(
    pl.kernel,
    out_type=jax.ShapeDtypeStruct((T, N_FEATURES), jnp.float32),
    mesh=plsc.VectorSubcoreMesh(core_axis_name="c", subcore_axis_name="s"),
    scratch_types=[
        pltpu.VMEM((STAGE_ROWS, N_FEATURES), jnp.float32),
        pltpu.VMEM((STAGE_ROWS,), jnp.float32),
        pltpu.VMEM((WIN_PER_W,), jnp.float32),
        pltpu.VMEM((WINDOW, WIN_PER_W), jnp.float32),
        pltpu.VMEM((WIN_PER_W, N_FEATURES), jnp.float32),
        pltpu.SemaphoreType.DMA,
        pltpu.SemaphoreType.DMA,
    ],
)(_sc_agg_body)


def kernel(ori_feats, W, attn_l, attn_r, bias):
    hb, el, er = pl.pallas_call(
        _dense_body,
        out_shape=[
            jax.ShapeDtypeStruct((HP_ROWS, N_FEATURES), jnp.float32),
            jax.ShapeDtypeStruct((HP_ROWS,), jnp.float32),
            jax.ShapeDtypeStruct((HP_ROWS,), jnp.float32),
        ],
        in_specs=[pl.BlockSpec(memory_space=pltpu.VMEM)] * 5,
        out_specs=[pl.BlockSpec(memory_space=pltpu.VMEM)] * 3,
    )(ori_feats, W, attn_l.reshape(1, N_FEATURES), attn_r.reshape(1, N_FEATURES),
      bias.reshape(1, N_FEATURES))

    out = _sc_agg(hb, el, er)
    return out[:, None, :]
